# baseline (speedup 1.0000x reference)
import functools

import jax
import jax.numpy as jnp
from jax import lax
from jax.experimental import pallas as pl
from jax.experimental.pallas import tpu as pltpu
from jax.experimental.pallas import tpu_sc as plsc

NC = 2
NS = 16
NW = NC * NS

CR = 16
NB = 2


def _gather_kernel(rows_w, n_chunks, D, data_hbm, table_hbm, out_hbm,
                   idx_v, rows_a, rows_b, gsem, ssem):
    B0, S = data_hbm.shape
    bufs = (rows_a, rows_b)
    wid = lax.axis_index("s") * NC + lax.axis_index("c")
    base = wid * rows_w

    pltpu.sync_copy(data_hbm.at[pl.ds(base, rows_w)], idx_v)

    def outer(o, carry):
        g0 = o * NB
        for b in range(NB):
            off = (g0 + b) * CR
            for r in range(CR):
                pltpu.async_copy(table_hbm.at[idx_v.at[off + r]],
                                 bufs[b].at[r], gsem)
        for b in range(NB):
            off = (g0 + b) * CR
            for r in range(CR):
                pltpu.make_async_copy(table_hbm.at[idx_v.at[off + r]],
                                      bufs[b].at[r], gsem).wait()
            for r in range(CR):
                pltpu.async_copy(bufs[b].at[r],
                                 out_hbm.at[base + off + r], ssem)
        for b in range(NB):
            off = (g0 + b) * CR
            for r in range(CR):
                pltpu.make_async_copy(bufs[b].at[r],
                                      out_hbm.at[base + off + r],
                                      ssem).wait()
        return carry

    lax.fori_loop(0, n_chunks // NB, outer, 0)


def kernel(data, table):
    B0, S = data.shape
    V, D = table.shape

    rows_w = B0 // NW
    n_chunks = rows_w // CR
    assert rows_w * NW == B0 and n_chunks * CR == rows_w
    assert n_chunks % NB == 0

    mesh = plsc.VectorSubcoreMesh(core_axis_name="c", subcore_axis_name="s")
    run = functools.partial(
        pl.kernel,
        out_type=jax.ShapeDtypeStruct((B0, S, D), jnp.float32),
        mesh=mesh,
        scratch_types=[
            pltpu.VMEM((rows_w, S), jnp.int32),
            pltpu.VMEM((CR, S, D), jnp.float32),
            pltpu.VMEM((CR, S, D), jnp.float32),
            pltpu.SemaphoreType.DMA,
            pltpu.SemaphoreType.DMA,
        ],
        compiler_params=pltpu.CompilerParams(use_tc_tiling_on_sc=False),
    )(functools.partial(_gather_kernel, rows_w, n_chunks, D))
    return run(data, table)


# speedup vs baseline: 1.0039x; 1.0039x over previous
import functools

import jax
import jax.numpy as jnp
from jax import lax
from jax.experimental import pallas as pl
from jax.experimental.pallas import tpu as pltpu
from jax.experimental.pallas import tpu_sc as plsc

NC = 2
NS = 16
NW = NC * NS

SP = 32
NB = 2


def _gather_kernel(bw, S, D, dataT_hbm, table_hbm, out_hbm,
                   idx_v, rows_a, rows_b, gsem, ssem):
    bufs = (rows_a, rows_b)
    wid = lax.axis_index("s") * NC + lax.axis_index("c")
    base = wid * bw

    pltpu.sync_copy(dataT_hbm.at[:, pl.ds(base, bw)], idx_v)

    def gather(s, b):
        pltpu.async_copy(table_hbm.at[idx_v.at[s]], bufs[b], gsem)

    def gwait(s, b):
        pltpu.make_async_copy(table_hbm.at[idx_v.at[s]], bufs[b],
                              gsem).wait()

    def store(s, b):
        pltpu.async_copy(bufs[b], out_hbm.at[pl.ds(base, bw), s], ssem)

    def swait(s, b):
        pltpu.make_async_copy(bufs[b], out_hbm.at[pl.ds(base, bw), s],
                              ssem).wait()

    for b in range(NB):
        gather(b, b)
    tail = []
    for s in range(S):
        b = s % NB
        gwait(s, b)
        store(s, b)
        if s + NB < S:
            swait(s, b)
            gather(s + NB, b)
        else:
            tail.append((s, b))
    for s, b in tail:
        swait(s, b)


def kernel(data, table):
    B0, S = data.shape
    V, D = table.shape

    bw = B0 // NW
    assert bw * NW == B0

    dataT_p = jnp.pad(data.T, ((0, SP - S), (0, 0)))

    mesh = plsc.VectorSubcoreMesh(core_axis_name="c", subcore_axis_name="s")
    run = functools.partial(
        pl.kernel,
        out_type=jax.ShapeDtypeStruct((B0, S, D), jnp.float32),
        mesh=mesh,
        scratch_types=[
            pltpu.VMEM((SP, bw), jnp.int32),
            pltpu.VMEM((bw, D), jnp.float32),
            pltpu.VMEM((bw, D), jnp.float32),
            pltpu.SemaphoreType.DMA,
            pltpu.SemaphoreType.DMA,
        ],
        compiler_params=pltpu.CompilerParams(use_tc_tiling_on_sc=False),
    )(functools.partial(_gather_kernel, bw, S, D))
    return run(dataT_p, table)
